# baseline (speedup 1.0000x reference)
import functools

import jax
import jax.numpy as jnp
from jax import lax
from jax.experimental import pallas as pl
from jax.experimental.pallas import tpu as pltpu
from jax.experimental.pallas import tpu_sc as plsc

N_USERS = 1000000
N_MOVIES = 100000
N_FACTORS = 64
BATCH = 16384
NC = 2
NS = 16
NW = NC * NS
BPW = BATCH // NW
LANES = 16
NGROUP = BPW // LANES
NCH = BPW // 128
DBLK = 4
NBLK = N_FACTORS // DBLK


def _cf_body(uids, mids, utab, mtab, ubtab, mbtab, gbias, out,
             uidx_v, midx_v, uidx_c, midx_c, urows_t, mrows_t,
             ub_v, mb_v, gb_v, out_v, sem):
    wid = lax.axis_index("s") * NC + lax.axis_index("c")
    base = wid * BPW

    pltpu.sync_copy(uids.at[pl.ds(base, BPW)], uidx_v)
    pltpu.sync_copy(mids.at[pl.ds(base, BPW)], midx_v)
    for j in range(NCH):
        pltpu.sync_copy(uids.at[pl.ds(base + j * 128, 128)], uidx_c[j])
        pltpu.sync_copy(mids.at[pl.ds(base + j * 128, 128)], midx_c[j])
    pltpu.sync_copy(gbias, gb_v.at[pl.ds(0, 1)])

    bias_copies = [
        pltpu.async_copy(ubtab.at[uidx_v], ub_v, sem),
        pltpu.async_copy(mbtab.at[midx_v], mb_v, sem),
    ]

    def block_streams(blk, issue):
        for dd in range(DBLK):
            d = blk * DBLK + dd
            uoff = pl.multiple_of(d * N_USERS, 8)
            moff = pl.multiple_of(d * N_MOVIES, 8)
            usrc = utab.at[pl.ds(uoff, N_USERS)]
            msrc = mtab.at[pl.ds(moff, N_MOVIES)]
            for j in range(NCH):
                sl = pl.ds(j * 128, 128)
                if issue:
                    pltpu.async_copy(usrc.at[uidx_c[j]],
                                     urows_t.at[d, sl], sem)
                    pltpu.async_copy(msrc.at[midx_c[j]],
                                     mrows_t.at[d, sl], sem)
                else:
                    pltpu.make_async_copy(usrc.at[uidx_c[j]],
                                          urows_t.at[d, sl], sem).wait()
                    pltpu.make_async_copy(msrc.at[midx_c[j]],
                                          mrows_t.at[d, sl], sem).wait()

    def pipe_body(i, carry):
        @pl.when(i < NBLK)
        def _():
            block_streams(i, True)

        @pl.when(i >= 1)
        def _():
            block_streams(i - 1, False)
        return carry

    lax.fori_loop(0, NBLK + 1, pipe_body, 0)
    for c in bias_copies:
        c.wait()

    gb_vec = jnp.zeros((LANES,), jnp.float32) + gb_v[...][0]

    def group_body(g, carry):
        sl = pl.ds(g * LANES, LANES)

        def dot_body(d, acc):
            return acc + urows_t[d, sl] * mrows_t[d, sl]

        acc = lax.fori_loop(0, N_FACTORS, dot_body,
                            jnp.zeros((LANES,), jnp.float32), unroll=8)
        r = acc + ub_v[sl] + mb_v[sl] + gb_vec
        out_v[sl] = 1.0 / (1.0 + jnp.exp(-r))
        return carry

    lax.fori_loop(0, NGROUP, group_body, 0)
    pltpu.sync_copy(out_v, out.at[pl.ds(base, BPW)])


@jax.jit
def _cf_call(uids, mids, utab, mtab, ubtab, mbtab, gbias):
    mesh = plsc.VectorSubcoreMesh(core_axis_name="c", subcore_axis_name="s")
    return pl.kernel(
        _cf_body,
        out_type=jax.ShapeDtypeStruct((BATCH,), jnp.float32),
        mesh=mesh,
        scratch_types=[
            pltpu.VMEM((BPW,), jnp.int32),
            pltpu.VMEM((BPW,), jnp.int32),
            [pltpu.VMEM((128,), jnp.int32) for _ in range(NCH)],
            [pltpu.VMEM((128,), jnp.int32) for _ in range(NCH)],
            pltpu.VMEM((N_FACTORS, BPW), jnp.float32),
            pltpu.VMEM((N_FACTORS, BPW), jnp.float32),
            pltpu.VMEM((BPW,), jnp.float32),
            pltpu.VMEM((BPW,), jnp.float32),
            pltpu.VMEM((LANES,), jnp.float32),
            pltpu.VMEM((BPW,), jnp.float32),
            pltpu.SemaphoreType.DMA,
        ],
    )(uids, mids, utab, mtab, ubtab, mbtab, gbias)


def kernel(user_ids, movie_ids, user_table, movie_table, user_bias_table,
           movie_bias_table, global_bias):
    return _cf_call(user_ids.astype(jnp.int32), movie_ids.astype(jnp.int32),
                    user_table.T.reshape(-1), movie_table.T.reshape(-1),
                    user_bias_table.reshape(-1),
                    movie_bias_table.reshape(-1), global_bias)
